# baseline (speedup 1.0000x reference)
import jax
import jax.numpy as jnp
from jax.experimental import pallas as pl
from jax.experimental.pallas import tpu as pltpu

_B = 16384
_K = 64
_CH = 2048
_NC = _B // _CH
_NBUF = 3


def _in_cp(src_hbm, buf, sem, c, b):
    return pltpu.make_async_copy(
        src_hbm.at[:, pl.ds(c * _CH, _CH)], buf.at[b], sem.at[b])


def _out_cp(buf, dst_hbm, sem, c, b):
    return pltpu.make_async_copy(
        buf.at[b], dst_hbm.at[:, pl.ds(c * _CH, _CH)], sem.at[b])


def _rowdot_body(gu_hbm, gi_hbm, xui_ref, guo_hbm, gio_hbm,
                 ubuf, vbuf, isem_u, isem_v, osem_u, osem_v):
    _in_cp(gu_hbm, ubuf, isem_u, 0, 0).start()
    _in_cp(gi_hbm, vbuf, isem_v, 0, 0).start()
    for c in range(_NC):
        b = c % _NBUF
        if c + 1 < _NC:
            bn = (c + 1) % _NBUF
            if c + 1 >= _NBUF:
                _out_cp(ubuf, guo_hbm, osem_u, c + 1 - _NBUF, bn).wait()
                _out_cp(vbuf, gio_hbm, osem_v, c + 1 - _NBUF, bn).wait()
            _in_cp(gu_hbm, ubuf, isem_u, c + 1, bn).start()
            _in_cp(gi_hbm, vbuf, isem_v, c + 1, bn).start()
        _in_cp(gu_hbm, ubuf, isem_u, c, b).wait()
        _in_cp(gi_hbm, vbuf, isem_v, c, b).wait()
        u = ubuf[b]
        v = vbuf[b]
        xui_ref[pl.ds(c * _CH, _CH)] = jnp.sum(u * v, axis=0)
        _out_cp(ubuf, guo_hbm, osem_u, c, b).start()
        _out_cp(vbuf, gio_hbm, osem_v, c, b).start()
    for c in range(max(_NC - _NBUF, 0), _NC):
        b = c % _NBUF
        _out_cp(ubuf, guo_hbm, osem_u, c, b).wait()
        _out_cp(vbuf, gio_hbm, osem_v, c, b).wait()


def kernel(gu, gi):
    gut = gu.T
    git = gi.T
    xui, guo_t, gio_t = pl.pallas_call(
        _rowdot_body,
        in_specs=[
            pl.BlockSpec(memory_space=pltpu.MemorySpace.HBM),
            pl.BlockSpec(memory_space=pltpu.MemorySpace.HBM),
        ],
        out_specs=[
            pl.BlockSpec(memory_space=pltpu.MemorySpace.VMEM),
            pl.BlockSpec(memory_space=pltpu.MemorySpace.HBM),
            pl.BlockSpec(memory_space=pltpu.MemorySpace.HBM),
        ],
        out_shape=[
            jax.ShapeDtypeStruct((_B,), jnp.float32),
            jax.ShapeDtypeStruct((_K, _B), jnp.float32),
            jax.ShapeDtypeStruct((_K, _B), jnp.float32),
        ],
        scratch_shapes=[
            pltpu.VMEM((_NBUF, _K, _CH), jnp.float32),
            pltpu.VMEM((_NBUF, _K, _CH), jnp.float32),
            pltpu.SemaphoreType.DMA((_NBUF,)),
            pltpu.SemaphoreType.DMA((_NBUF,)),
            pltpu.SemaphoreType.DMA((_NBUF,)),
            pltpu.SemaphoreType.DMA((_NBUF,)),
        ],
    )(gut, git)
    return (xui, guo_t.T, gio_t.T)


# speedup vs baseline: 1.1890x; 1.1890x over previous
import jax
import jax.numpy as jnp
from jax.experimental import pallas as pl
from jax.experimental.pallas import tpu as pltpu

_B = 16384
_K = 64
_CH = 4096
_NC = _B // _CH
_NBUF = 3


def _in_cp(src_hbm, buf, sem, c, b):
    return pltpu.make_async_copy(
        src_hbm.at[:, pl.ds(c * _CH, _CH)], buf.at[b], sem.at[b])


def _out_cp(buf, dst_hbm, sem, c, b):
    return pltpu.make_async_copy(
        buf.at[b], dst_hbm.at[:, pl.ds(c * _CH, _CH)], sem.at[b])


def _rowdot_body(gu_hbm, gi_hbm, xui_ref, guo_hbm, gio_hbm,
                 ubuf, vbuf, isem_u, isem_v, osem_u, osem_v):
    _in_cp(gu_hbm, ubuf, isem_u, 0, 0).start()
    _in_cp(gi_hbm, vbuf, isem_v, 0, 0).start()
    for c in range(_NC):
        b = c % _NBUF
        if c + 1 < _NC:
            bn = (c + 1) % _NBUF
            if c + 1 >= _NBUF:
                _out_cp(ubuf, guo_hbm, osem_u, c + 1 - _NBUF, bn).wait()
                _out_cp(vbuf, gio_hbm, osem_v, c + 1 - _NBUF, bn).wait()
            _in_cp(gu_hbm, ubuf, isem_u, c + 1, bn).start()
            _in_cp(gi_hbm, vbuf, isem_v, c + 1, bn).start()
        _in_cp(gu_hbm, ubuf, isem_u, c, b).wait()
        _in_cp(gi_hbm, vbuf, isem_v, c, b).wait()
        u = ubuf[b]
        v = vbuf[b]
        xui_ref[pl.ds(c * _CH, _CH)] = jnp.sum(u * v, axis=0)
        _out_cp(ubuf, guo_hbm, osem_u, c, b).start()
        _out_cp(vbuf, gio_hbm, osem_v, c, b).start()
    for c in range(max(_NC - _NBUF, 0), _NC):
        b = c % _NBUF
        _out_cp(ubuf, guo_hbm, osem_u, c, b).wait()
        _out_cp(vbuf, gio_hbm, osem_v, c, b).wait()


def kernel(gu, gi):
    gut = gu.T
    git = gi.T
    xui, guo_t, gio_t = pl.pallas_call(
        _rowdot_body,
        in_specs=[
            pl.BlockSpec(memory_space=pltpu.MemorySpace.HBM),
            pl.BlockSpec(memory_space=pltpu.MemorySpace.HBM),
        ],
        out_specs=[
            pl.BlockSpec(memory_space=pltpu.MemorySpace.VMEM),
            pl.BlockSpec(memory_space=pltpu.MemorySpace.HBM),
            pl.BlockSpec(memory_space=pltpu.MemorySpace.HBM),
        ],
        out_shape=[
            jax.ShapeDtypeStruct((_B,), jnp.float32),
            jax.ShapeDtypeStruct((_K, _B), jnp.float32),
            jax.ShapeDtypeStruct((_K, _B), jnp.float32),
        ],
        scratch_shapes=[
            pltpu.VMEM((_NBUF, _K, _CH), jnp.float32),
            pltpu.VMEM((_NBUF, _K, _CH), jnp.float32),
            pltpu.SemaphoreType.DMA((_NBUF,)),
            pltpu.SemaphoreType.DMA((_NBUF,)),
            pltpu.SemaphoreType.DMA((_NBUF,)),
            pltpu.SemaphoreType.DMA((_NBUF,)),
        ],
    )(gut, git)
    return (xui, guo_t.T, gio_t.T)


# speedup vs baseline: 1.3693x; 1.1517x over previous
import jax
import jax.numpy as jnp
from jax.experimental import pallas as pl
from jax.experimental.pallas import tpu as pltpu

_B = 16384
_K = 64
_CH = 8192
_NC = _B // _CH
_NBUF = 2


def _in_cp(src_hbm, buf, sem, c, b):
    return pltpu.make_async_copy(
        src_hbm.at[:, pl.ds(c * _CH, _CH)], buf.at[b], sem.at[b])


def _out_cp(buf, dst_hbm, sem, c, b):
    return pltpu.make_async_copy(
        buf.at[b], dst_hbm.at[:, pl.ds(c * _CH, _CH)], sem.at[b])


def _rowdot_body(gu_hbm, gi_hbm, xui_ref, guo_hbm, gio_hbm,
                 ubuf, vbuf, isem_u, isem_v, osem_u, osem_v):
    _in_cp(gu_hbm, ubuf, isem_u, 0, 0).start()
    _in_cp(gi_hbm, vbuf, isem_v, 0, 0).start()
    for c in range(_NC):
        b = c % _NBUF
        if c + 1 < _NC:
            bn = (c + 1) % _NBUF
            if c + 1 >= _NBUF:
                _out_cp(ubuf, guo_hbm, osem_u, c + 1 - _NBUF, bn).wait()
                _out_cp(vbuf, gio_hbm, osem_v, c + 1 - _NBUF, bn).wait()
            _in_cp(gu_hbm, ubuf, isem_u, c + 1, bn).start()
            _in_cp(gi_hbm, vbuf, isem_v, c + 1, bn).start()
        _in_cp(gu_hbm, ubuf, isem_u, c, b).wait()
        _in_cp(gi_hbm, vbuf, isem_v, c, b).wait()
        u = ubuf[b]
        v = vbuf[b]
        xui_ref[pl.ds(c * _CH, _CH)] = jnp.sum(u * v, axis=0)
        _out_cp(ubuf, guo_hbm, osem_u, c, b).start()
        _out_cp(vbuf, gio_hbm, osem_v, c, b).start()
    for c in range(max(_NC - _NBUF, 0), _NC):
        b = c % _NBUF
        _out_cp(ubuf, guo_hbm, osem_u, c, b).wait()
        _out_cp(vbuf, gio_hbm, osem_v, c, b).wait()


def kernel(gu, gi):
    gut = gu.T
    git = gi.T
    xui, guo_t, gio_t = pl.pallas_call(
        _rowdot_body,
        in_specs=[
            pl.BlockSpec(memory_space=pltpu.MemorySpace.HBM),
            pl.BlockSpec(memory_space=pltpu.MemorySpace.HBM),
        ],
        out_specs=[
            pl.BlockSpec(memory_space=pltpu.MemorySpace.VMEM),
            pl.BlockSpec(memory_space=pltpu.MemorySpace.HBM),
            pl.BlockSpec(memory_space=pltpu.MemorySpace.HBM),
        ],
        out_shape=[
            jax.ShapeDtypeStruct((_B,), jnp.float32),
            jax.ShapeDtypeStruct((_K, _B), jnp.float32),
            jax.ShapeDtypeStruct((_K, _B), jnp.float32),
        ],
        scratch_shapes=[
            pltpu.VMEM((_NBUF, _K, _CH), jnp.float32),
            pltpu.VMEM((_NBUF, _K, _CH), jnp.float32),
            pltpu.SemaphoreType.DMA((_NBUF,)),
            pltpu.SemaphoreType.DMA((_NBUF,)),
            pltpu.SemaphoreType.DMA((_NBUF,)),
            pltpu.SemaphoreType.DMA((_NBUF,)),
        ],
    )(gut, git)
    return (xui, guo_t.T, gio_t.T)


# speedup vs baseline: 1.5200x; 1.1100x over previous
import jax
import jax.numpy as jnp
from jax.experimental import pallas as pl

_B = 16384
_K = 64
_BLK = 8192


def _rowdot_body(gu_ref, gi_ref, xui_ref, guo_ref, gio_ref):
    u = gu_ref[...]
    v = gi_ref[...]
    xui_ref[...] = jnp.sum(u * v, axis=0)
    guo_ref[...] = u
    gio_ref[...] = v


def kernel(gu, gi):
    gut = gu.T
    git = gi.T
    xui, guo_t, gio_t = pl.pallas_call(
        _rowdot_body,
        grid=(_B // _BLK,),
        in_specs=[
            pl.BlockSpec((_K, _BLK), lambda i: (0, i)),
            pl.BlockSpec((_K, _BLK), lambda i: (0, i)),
        ],
        out_specs=[
            pl.BlockSpec((_BLK,), lambda i: (i,)),
            pl.BlockSpec((_K, _BLK), lambda i: (0, i)),
            pl.BlockSpec((_K, _BLK), lambda i: (0, i)),
        ],
        out_shape=[
            jax.ShapeDtypeStruct((_B,), jnp.float32),
            jax.ShapeDtypeStruct((_K, _B), jnp.float32),
            jax.ShapeDtypeStruct((_K, _B), jnp.float32),
        ],
    )(gut, git)
    return (xui, guo_t.T, gio_t.T)
